# baseline (speedup 1.0000x reference)
import jax
import jax.numpy as jnp
from jax import lax
from jax.experimental import pallas as pl
from jax.experimental.pallas import tpu as pltpu
from jax.experimental.pallas import tpu_sc as plsc

NC = 2
NS = 16
NW = NC * NS

D = 33
CHUNK = 128
B_TOTAL = 16384 * 10
PER_W = B_TOTAL // NW
NCHUNK = PER_W // CHUNK


def _body(idx_hbm, table_hbm, out_hbm, idx_v, row_v, gsem):
    wid = lax.axis_index("s") * NC + lax.axis_index("c")
    pltpu.sync_copy(idx_hbm.at[wid], idx_v)
    base = wid * PER_W

    def step(j, carry):
        pltpu.async_copy(table_hbm.at[idx_v.at[j]], row_v, gsem).wait()
        pltpu.sync_copy(row_v, out_hbm.at[pl.ds(base + j * CHUNK, CHUNK)])
        return carry

    lax.fori_loop(0, NCHUNK, step, 0)


def kernel(idx, embedding):
    idx3 = idx.reshape(NW, NCHUNK, CHUNK)
    mesh = plsc.VectorSubcoreMesh(
        core_axis_name="c", subcore_axis_name="s", num_cores=NC, num_subcores=NS
    )
    out = pl.kernel(
        _body,
        out_type=jax.ShapeDtypeStruct((B_TOTAL, D), jnp.float32),
        mesh=mesh,
        scratch_types=[
            pltpu.VMEM((NCHUNK, CHUNK), jnp.int32),
            pltpu.VMEM((CHUNK, D), jnp.float32),
            pltpu.SemaphoreType.DMA,
        ],
        compiler_params=pltpu.CompilerParams(use_tc_tiling_on_sc=False),
    )(idx3, embedding)
    return out.reshape(idx.shape[0], idx.shape[1], D)


# speedup vs baseline: 1.0200x; 1.0200x over previous
import jax
import jax.numpy as jnp
from jax import lax
from jax.experimental import pallas as pl
from jax.experimental.pallas import tpu as pltpu
from jax.experimental.pallas import tpu_sc as plsc

NC = 2
NS = 16
NW = NC * NS

D = 33
CHUNK = 128
B_TOTAL = 16384 * 10
PER_W = B_TOTAL // NW
NCHUNK = PER_W // CHUNK
NBUF = 10
L1 = NBUF // 2
NROUND = NCHUNK // NBUF


def _body(idx_hbm, table_hbm, out_hbm, idx_v, rows_v, gsem, osem):
    wid = lax.axis_index("s") * NC + lax.axis_index("c")
    pltpu.sync_copy(idx_hbm.at[wid], idx_v)
    base = wid * PER_W

    def gather(j, s):
        pltpu.async_copy(table_hbm.at[idx_v.at[j]], rows_v.at[s], gsem.at[s])

    def out_copy(j, s):
        pltpu.async_copy(
            rows_v.at[s], out_hbm.at[pl.ds(base + j * CHUNK, CHUNK)], osem.at[s]
        )

    def wait_gather(s):
        pltpu.make_async_copy(
            table_hbm.at[idx_v.at[0]], rows_v.at[s], gsem.at[s]
        ).wait()

    def wait_out(s):
        pltpu.make_async_copy(
            rows_v.at[s], out_hbm.at[pl.ds(base, CHUNK)], osem.at[s]
        ).wait()

    for b in range(L1):
        gather(b, b)

    def round_fn(r, carry):
        for b in range(NBUF):
            j = r * NBUF + b
            s_new = (b + L1) % NBUF
            j_new = j + L1
            if b < L1:
                @pl.when(r >= 1)
                def _():
                    wait_out(s_new)
                    gather(j_new, s_new)

                @pl.when(r == 0)
                def _():
                    gather(j_new, s_new)
            else:
                @pl.when(r < NROUND - 1)
                def _():
                    wait_out(s_new)
                    gather(j_new, s_new)

            wait_gather(b)
            out_copy(j, b)
        return carry

    lax.fori_loop(0, NROUND, round_fn, 0)

    for b in range(NBUF):
        wait_out(b)


def kernel(idx, embedding):
    idx3 = idx.reshape(NW, NCHUNK, CHUNK)
    mesh = plsc.VectorSubcoreMesh(
        core_axis_name="c", subcore_axis_name="s", num_cores=NC, num_subcores=NS
    )
    out = pl.kernel(
        _body,
        out_type=jax.ShapeDtypeStruct((B_TOTAL, D), jnp.float32),
        mesh=mesh,
        scratch_types=[
            pltpu.VMEM((NCHUNK, CHUNK), jnp.int32),
            pltpu.VMEM((NBUF, CHUNK, D), jnp.float32),
            pltpu.SemaphoreType.DMA((NBUF,)),
            pltpu.SemaphoreType.DMA((NBUF,)),
        ],
        compiler_params=pltpu.CompilerParams(use_tc_tiling_on_sc=False),
    )(idx3, embedding)
    return out.reshape(idx.shape[0], idx.shape[1], D)
